# baseline (speedup 1.0000x reference)
import functools

import jax
import jax.numpy as jnp
from jax import lax
from jax.experimental import pallas as pl
from jax.experimental.pallas import tpu as pltpu
import jax.experimental.pallas.tpu_sc as plsc

N = 10000
E = 320000
NFEAT = 128
NHID = 128
NCLASS = 40
ALPHA = 0.1
NLAYERS = 2

DP = 40
NC = 2
NS = 16
NW = NC * NS
EPW = E // NW
CH = 128
NGRP = E // CH
GPW = NGRP // NW
GEXTRA = NGRP - GPW * NW
GPW_MAX = GPW + 1
NP = 10240
RPT = NP // NS


_RB = 1000


def _linear_body(x_ref, w1_ref, b1_ref, w2_ref, b2_ref, o_ref):
    h = jnp.dot(x_ref[...], w1_ref[...], preferred_element_type=jnp.float32)
    h = jnp.maximum(h + b1_ref[...], 0.0)
    o_ref[...] = (
        jnp.dot(h, w2_ref[...], preferred_element_type=jnp.float32) + b2_ref[...]
    )


def _linear(x, W1, b1, W2p, b2p):
    return pl.pallas_call(
        _linear_body,
        grid=(N // _RB,),
        in_specs=[
            pl.BlockSpec((_RB, NFEAT), lambda i: (i, 0)),
            pl.BlockSpec((NFEAT, NHID), lambda i: (0, 0)),
            pl.BlockSpec((1, NHID), lambda i: (0, 0)),
            pl.BlockSpec((NHID, DP), lambda i: (0, 0)),
            pl.BlockSpec((1, DP), lambda i: (0, 0)),
        ],
        out_specs=pl.BlockSpec((_RB, DP), lambda i: (i, 0)),
        out_shape=jax.ShapeDtypeStruct((N, DP), jnp.float32),
    )(x, W1, b1, W2p, b2p)


_MESH = plsc.VectorSubcoreMesh(
    core_axis_name="c", subcore_axis_name="s", num_cores=NC, num_subcores=NS
)


@functools.partial(
    pl.kernel,
    out_type=jax.ShapeDtypeStruct((NC, NP, DP), jnp.float32),
    mesh=_MESH,
    scratch_types=[
        pltpu.VMEM((GPW_MAX, 2, 128), jnp.int32),
        pltpu.VMEM((GPW_MAX * 128,), jnp.float32),
        [pltpu.VMEM((CH, DP), jnp.float32) for _ in range(5)],
        [pltpu.SemaphoreType.DMA for _ in range(5)],
        [pltpu.SemaphoreType.DMA for _ in range(5)],
        pltpu.VMEM_SHARED((NP, DP), jnp.float32),
    ],
    compiler_params=pltpu.CompilerParams(use_tc_tiling_on_sc=False),
)
def _spmm(z_hbm, ev_hbm, w_hbm, zero_hbm, out_hbm,
          evv, wv, bufs, gsem, ssem, acc):
    cid = lax.axis_index("c")
    sid = lax.axis_index("s")
    wid = cid * NS + sid
    NB = 5

    base = wid * GPW + jnp.minimum(wid, GEXTRA)
    myn = GPW + jnp.where(wid < GEXTRA, 1, 0)
    base79 = jnp.minimum(base, NGRP - GPW_MAX)
    d = base - base79

    pltpu.sync_copy(zero_hbm, acc.at[pl.ds(sid * RPT, RPT)])

    pltpu.sync_copy(ev_hbm.at[pl.ds(base79, GPW_MAX)], evv)
    pltpu.sync_copy(w_hbm.at[pl.ds(base79 * CH, GPW_MAX * CH)], wv)

    plsc.subcore_barrier()

    def gather_start(j, b):
        pltpu.async_copy(z_hbm.at[evv.at[j + d, 1]], bufs[b], gsem[b])

    def gather_wait(j, b):
        pltpu.make_async_copy(z_hbm.at[evv.at[j + d, 1]], bufs[b], gsem[b]).wait()

    def scat_start(j, b):
        pltpu.async_copy(bufs[b], acc.at[evv.at[j + d, 0]], ssem[b], add=True)

    def scat_wait(j, b):
        pltpu.make_async_copy(bufs[b], acc.at[evv.at[j + d, 0]], ssem[b]).wait()

    lane = lax.iota(jnp.int32, 16)

    def mul(j, b):
        for g in range(CH // 16):
            w16 = wv[pl.ds((j + d) * CH + g * 16, 16)]
            for u in range(16):
                e = g * 16 + u
                w_e = w16[u]
                w_hi = jnp.where(lane < 8, 1.0, w_e)
                bufs[b][e, pl.ds(0, 16)] = bufs[b][e, pl.ds(0, 16)] * w_e
                bufs[b][e, pl.ds(16, 16)] = bufs[b][e, pl.ds(16, 16)] * w_e
                bufs[b][e, pl.ds(24, 16)] = bufs[b][e, pl.ds(24, 16)] * w_hi

    gather_start(0, 0)
    gather_start(1, 1)
    gather_start(2, 2)

    def group_body(g, carry):
        for s in range(NB):
            k = g * NB + s
            ahead = k + 3
            b = s
            b2 = (s + 3) % NB

            @pl.when(jnp.logical_and(k >= 2, k < myn + 2))
            def _():
                scat_wait(k - 2, b2)

            @pl.when(ahead < myn)
            def _():
                gather_start(ahead, b2)

            @pl.when(k < myn)
            def _():
                gather_wait(k, b)
                mul(k, b)
                scat_start(k, b)
        return carry

    lax.fori_loop(0, (GPW_MAX + NB) // NB, group_body, 0, unroll=False)

    @pl.when(myn == GPW_MAX)
    def _():
        scat_wait(GPW_MAX - 1, (GPW_MAX - 1) % NB)

    plsc.subcore_barrier()

    pltpu.sync_copy(acc.at[pl.ds(sid * RPT, RPT)],
                    out_hbm.at[cid, pl.ds(sid * RPT, RPT)])


_CB = 2000

PROWS = NP * DP // 128
ZROWS = N * DP // 128
_CBP = 768


def _combine_body(p_ref, hp_ref, o_ref):
    o_ref[...] = (1.0 - ALPHA) * (p_ref[0] + p_ref[1]) + ALPHA * hp_ref[...]


def _combine(p_packed, hp):
    return pl.pallas_call(
        _combine_body,
        grid=((ZROWS + _CBP - 1) // _CBP,),
        in_specs=[
            pl.BlockSpec((NC, _CBP, 128), lambda i: (0, i, 0)),
            pl.BlockSpec((_CBP, 128), lambda i: (i, 0)),
        ],
        out_specs=pl.BlockSpec((_CBP, 128), lambda i: (i, 0)),
        out_shape=jax.ShapeDtypeStruct((ZROWS, 128), jnp.float32),
    )(p_packed, hp)


def _final_body(p_ref, h_ref, o_ref):
    t = (1.0 - ALPHA) * (p_ref[0] + p_ref[1]) + ALPHA * h_ref[...]
    m = jnp.max(t, axis=1, keepdims=True)
    s = jnp.sum(jnp.exp(t - m), axis=1, keepdims=True)
    o_ref[...] = t - m - jnp.log(s)


def _final(p, h):
    return pl.pallas_call(
        _final_body,
        grid=(N // _CB,),
        in_specs=[
            pl.BlockSpec((NC, _CB, DP), lambda i: (0, i, 0)),
            pl.BlockSpec((_CB, DP), lambda i: (i, 0)),
        ],
        out_specs=pl.BlockSpec((_CB, NCLASS), lambda i: (i, 0)),
        out_shape=jax.ShapeDtypeStruct((N, NCLASS), jnp.float32),
    )(p, h)


def kernel(x, edge_index, edge_weight, W1, b1, W2, b2):
    ev = edge_index.astype(jnp.int32).reshape(2, NGRP, CH).transpose(1, 0, 2)

    h = _linear(x, W1, b1.reshape(1, NHID), W2, b2.reshape(1, NCLASS))
    hp = h.reshape(ZROWS, 128)
    zeros = jnp.zeros((RPT, DP), jnp.float32)

    z = hp.reshape(N, DP)
    for r in range(NLAYERS):
        p = _spmm(z, ev, edge_weight, zeros)
        if r < NLAYERS - 1:
            zp = _combine(p.reshape(NC, PROWS, 128), hp)
            z = zp.reshape(N, DP)
    return _final(p, h)


# speedup vs baseline: 1.2176x; 1.2176x over previous
import functools

import jax
import jax.numpy as jnp
from jax import lax
from jax.experimental import pallas as pl
from jax.experimental.pallas import tpu as pltpu
import jax.experimental.pallas.tpu_sc as plsc

N = 10000
E = 320000
NFEAT = 128
NHID = 128
NCLASS = 40
ALPHA = 0.1
NLAYERS = 2

DP = 40
NC = 2
NS = 16
NW = NC * NS
EPW = E // NW
CH = 80
NCHUNK = EPW // CH
NP = 10240
RPT = NP // NS


_RB = 1000


def _linear_body(x_ref, w1_ref, b1_ref, w2_ref, b2_ref, o_ref):
    h = jnp.dot(x_ref[...], w1_ref[...], preferred_element_type=jnp.float32)
    h = jnp.maximum(h + b1_ref[...], 0.0)
    o_ref[...] = (
        jnp.dot(h, w2_ref[...], preferred_element_type=jnp.float32) + b2_ref[...]
    )


def _linear(x, W1, b1, W2p, b2p):
    return pl.pallas_call(
        _linear_body,
        grid=(N // _RB,),
        in_specs=[
            pl.BlockSpec((_RB, NFEAT), lambda i: (i, 0)),
            pl.BlockSpec((NFEAT, NHID), lambda i: (0, 0)),
            pl.BlockSpec((1, NHID), lambda i: (0, 0)),
            pl.BlockSpec((NHID, DP), lambda i: (0, 0)),
            pl.BlockSpec((1, DP), lambda i: (0, 0)),
        ],
        out_specs=pl.BlockSpec((_RB, DP), lambda i: (i, 0)),
        out_shape=jax.ShapeDtypeStruct((N, DP), jnp.float32),
    )(x, W1, b1, W2p, b2p)


_MESH = plsc.VectorSubcoreMesh(
    core_axis_name="c", subcore_axis_name="s", num_cores=NC, num_subcores=NS
)


@functools.partial(
    pl.kernel,
    out_type=jax.ShapeDtypeStruct((NC, NP, DP), jnp.float32),
    mesh=_MESH,
    scratch_types=[
        pltpu.VMEM((NCHUNK, CH), jnp.int32),
        pltpu.VMEM((NCHUNK, CH), jnp.int32),
        pltpu.VMEM((EPW,), jnp.float32),
        [pltpu.VMEM((CH, DP), jnp.float32) for _ in range(5)],
        [pltpu.SemaphoreType.DMA for _ in range(5)],
        [pltpu.SemaphoreType.DMA for _ in range(5)],
        pltpu.VMEM_SHARED((NP, DP), jnp.float32),
    ],
    compiler_params=pltpu.CompilerParams(use_tc_tiling_on_sc=False),
)
def _spmm(z_hbm, e_hbm, w_hbm, zero_hbm, out_hbm,
          colv, rowv, wv, bufs, gsem, ssem, acc):
    cid = lax.axis_index("c")
    sid = lax.axis_index("s")
    wid = cid * NS + sid
    NB = 5

    pltpu.sync_copy(zero_hbm, acc.at[pl.ds(sid * RPT, RPT)])

    pltpu.sync_copy(e_hbm.at[1, wid], colv)
    pltpu.sync_copy(e_hbm.at[0, wid], rowv)
    pltpu.sync_copy(w_hbm.at[pl.ds(wid * EPW, EPW)], wv)

    plsc.subcore_barrier()

    def gather_start(j, b):
        pltpu.async_copy(z_hbm.at[colv.at[j]], bufs[b], gsem[b])

    def gather_wait(j, b):
        pltpu.make_async_copy(z_hbm.at[colv.at[j]], bufs[b], gsem[b]).wait()

    def scat_start(j, b):
        pltpu.async_copy(bufs[b], acc.at[rowv.at[j]], ssem[b], add=True)

    def scat_wait(j, b):
        pltpu.make_async_copy(bufs[b], acc.at[rowv.at[j]], ssem[b]).wait()

    lane = lax.iota(jnp.int32, 16)

    def mul(j, b):
        for g in range(CH // 16):
            w16 = wv[pl.ds(j * CH + g * 16, 16)]
            for u in range(16):
                e = g * 16 + u
                w_e = w16[u]
                w_hi = jnp.where(lane < 8, 1.0, w_e)
                bufs[b][e, pl.ds(0, 16)] = bufs[b][e, pl.ds(0, 16)] * w_e
                bufs[b][e, pl.ds(16, 16)] = bufs[b][e, pl.ds(16, 16)] * w_e
                bufs[b][e, pl.ds(24, 16)] = bufs[b][e, pl.ds(24, 16)] * w_hi

    gather_start(0, 0)
    gather_start(1, 1)
    gather_start(2, 2)

    def group_body(g, carry):
        for s in range(NB):
            k = g * NB + s
            ahead = k + 3
            b = s
            b2 = (s + 3) % NB

            @pl.when(k >= 2)
            def _():
                scat_wait(k - 2, b2)

            @pl.when(ahead < NCHUNK)
            def _():
                gather_start(ahead, b2)

            gather_wait(k, b)
            mul(k, b)
            scat_start(k, b)
        return carry

    lax.fori_loop(0, NCHUNK // NB, group_body, 0, unroll=False)

    for k in range(NCHUNK - 2, NCHUNK):
        scat_wait(k, k % NB)

    plsc.subcore_barrier()

    pltpu.sync_copy(acc.at[pl.ds(sid * RPT, RPT)],
                    out_hbm.at[cid, pl.ds(sid * RPT, RPT)])


_CB = 2000

PROWS = NP * DP // 128
ZROWS = N * DP // 128
_CBP = 768


def _combine_body(p_ref, hp_ref, o_ref):
    o_ref[...] = (1.0 - ALPHA) * (p_ref[0] + p_ref[1]) + ALPHA * hp_ref[...]


def _combine(p_packed, hp):
    return pl.pallas_call(
        _combine_body,
        grid=((ZROWS + _CBP - 1) // _CBP,),
        in_specs=[
            pl.BlockSpec((NC, _CBP, 128), lambda i: (0, i, 0)),
            pl.BlockSpec((_CBP, 128), lambda i: (i, 0)),
        ],
        out_specs=pl.BlockSpec((_CBP, 128), lambda i: (i, 0)),
        out_shape=jax.ShapeDtypeStruct((ZROWS, 128), jnp.float32),
    )(p_packed, hp)


def _final_body(p_ref, h_ref, o_ref):
    t = (1.0 - ALPHA) * (p_ref[0] + p_ref[1]) + ALPHA * h_ref[...]
    m = jnp.max(t, axis=1, keepdims=True)
    s = jnp.sum(jnp.exp(t - m), axis=1, keepdims=True)
    o_ref[...] = t - m - jnp.log(s)


def _final(p, h):
    return pl.pallas_call(
        _final_body,
        grid=(N // _CB,),
        in_specs=[
            pl.BlockSpec((NC, _CB, DP), lambda i: (0, i, 0)),
            pl.BlockSpec((_CB, DP), lambda i: (i, 0)),
        ],
        out_specs=pl.BlockSpec((_CB, NCLASS), lambda i: (i, 0)),
        out_shape=jax.ShapeDtypeStruct((N, NCLASS), jnp.float32),
    )(p, h)


def kernel(x, edge_index, edge_weight, W1, b1, W2, b2):
    e4 = edge_index.astype(jnp.int32).reshape(2, NW, NCHUNK, CH)

    h = _linear(x, W1, b1.reshape(1, NHID), W2, b2.reshape(1, NCLASS))
    hp = h.reshape(ZROWS, 128)
    zeros = jnp.zeros((RPT, DP), jnp.float32)

    z = hp.reshape(N, DP)
    for r in range(NLAYERS):
        p = _spmm(z, e4, edge_weight, zeros)
        if r < NLAYERS - 1:
            zp = _combine(p.reshape(NC, PROWS, 128), hp)
            z = zp.reshape(N, DP)
    return _final(p, h)
